# baseline (speedup 1.0000x reference)
import functools

import jax
import jax.numpy as jnp
from jax import lax
from jax.experimental import pallas as pl
from jax.experimental.pallas import tpu as pltpu
from jax.experimental.pallas import tpu_sc as plsc

_N = 64 * 1024
_D = 64
_K = 512
_T = 1024
_G = _N // _T
_BETA = 0.25
_LOSS_SCALE = (1.0 + _BETA) / (_N * _D)

_NC, _NS = 2, 16
_NW = _NC * _NS
_TOK_PER_W = _N // _NW
_CH = 128
_NCHUNK = _TOK_PER_W // _CH
_SUPER = 4


def _vq_tc_body(z_ref, wm2_ref, idx_ref, loss_ref):
    zb = z_ref[0].T
    wm2 = wm2_ref[...]
    scores = lax.dot_general(zb, wm2, (((1,), (1,)), ((), ())),
                             preferred_element_type=jnp.float32)
    wsq = jnp.sum(wm2 * wm2, axis=1) * 0.25
    zsq = jnp.sum(zb * zb, axis=1, keepdims=True)
    dist = (zsq + scores) + wsq[None, :]
    minval = jnp.min(dist, axis=1, keepdims=True)
    lane = lax.broadcasted_iota(jnp.int32, dist.shape, 1)
    idx = jnp.min(jnp.where(dist == minval, lane, _K), axis=1,
                  keepdims=True)
    idx_ref[...] = idx.reshape(_T // _CH, _CH)
    step = pl.program_id(0)
    prev = loss_ref[...]
    acc = jnp.where(step == 0, jnp.zeros_like(prev), prev) + jnp.sum(minval)
    loss_ref[...] = jnp.where(step == pl.num_programs(0) - 1,
                              acc * _LOSS_SCALE, acc)


_vq_tc = pl.pallas_call(
    _vq_tc_body,
    grid=(_G,),
    in_specs=[
        pl.BlockSpec((1, _D, _T), lambda i: (i, 0, 0)),
        pl.BlockSpec((_K, _D), lambda i: (0, 0)),
    ],
    out_specs=[
        pl.BlockSpec((_T // _CH, _CH), lambda i: (i, 0)),
        pl.BlockSpec((1, 1), lambda i: (0, 0)),
    ],
    out_shape=[
        jax.ShapeDtypeStruct((_N // _CH, _CH), jnp.int32),
        jax.ShapeDtypeStruct((1, 1), jnp.float32),
    ],
)


def _sc_gather_body(idx_hbm, w_hbm, out_hbm, idx_v, w_sp, rows_v,
                    gsem, wsem):
    wid = lax.axis_index("s") * _NC + lax.axis_index("c")
    sid = lax.axis_index("s")

    @pl.when(sid == 0)
    def _():
        pltpu.sync_copy(w_hbm, w_sp)
    plsc.subcore_barrier()
    pltpu.sync_copy(idx_hbm.at[pl.ds(wid * _TOK_PER_W, _TOK_PER_W)], idx_v)

    wh = [None, None]
    for g in range(_NCHUNK):
        b = g % 2
        if wh[b] is not None:
            wh[b].wait()
        pltpu.async_copy(w_sp.at[idx_v.at[pl.ds(g * _CH, _CH)]],
                         rows_v.at[b], gsem).wait()
        wh[b] = pltpu.async_copy(
            rows_v.at[b],
            out_hbm.at[pl.ds(wid * _TOK_PER_W + g * _CH, _CH)],
            wsem.at[b])
    for b in range(2):
        if wh[b] is not None:
            wh[b].wait()


@functools.cache
def _sc_gather():
    return pl.kernel(
        _sc_gather_body,
        mesh=plsc.VectorSubcoreMesh(core_axis_name="c", subcore_axis_name="s"),
        out_type=jax.ShapeDtypeStruct((_N, _D), jnp.float32),
        compiler_params=pltpu.CompilerParams(use_tc_tiling_on_sc=False),
        scratch_types=[
            pltpu.VMEM((_TOK_PER_W,), jnp.int32),
            pltpu.VMEM_SHARED((_K, _D), jnp.float32),
            pltpu.VMEM((2, _CH, _D), jnp.float32),
            pltpu.SemaphoreType.DMA,
            pltpu.SemaphoreType.DMA((2,)),
        ],
    )


def kernel(z, W):
    zt = z.transpose(0, 2, 1)
    w_m2 = W * (-2.0)
    idx_rows, loss_arr = _vq_tc(zt, w_m2)
    z_q = _sc_gather()(idx_rows.reshape(-1), W)
    return (z_q.reshape(z.shape), loss_arr[0, 0],
            idx_rows.reshape(z.shape[0], z.shape[1]))


# speedup vs baseline: 1.1588x; 1.1588x over previous
import functools

import jax
import jax.numpy as jnp
from jax import lax
from jax.experimental import pallas as pl
from jax.experimental.pallas import tpu as pltpu
from jax.experimental.pallas import tpu_sc as plsc

_N = 64 * 1024
_D = 64
_K = 512
_T = 1024
_G = _N // _T
_BETA = 0.25
_LOSS_SCALE = (1.0 + _BETA) / (_N * _D)

_NC, _NS = 2, 16
_NW = _NC * _NS
_TOK_PER_W = _N // _NW
_CH = 128
_NCHUNK = _TOK_PER_W // _CH
_SUPER = 4


def _vq_tc_body(z_ref, wm2_ref, idx_ref, loss_ref):
    ztb = z_ref[0]
    wm2 = wm2_ref[...]
    scores = lax.dot_general(wm2, ztb, (((1,), (0,)), ((), ())),
                             preferred_element_type=jnp.float32)
    wsq = jnp.sum(wm2 * wm2, axis=1, keepdims=True) * 0.25
    zsq = jnp.sum(ztb * ztb, axis=0, keepdims=True)
    dist = (zsq + scores) + wsq
    minval = jnp.min(dist, axis=0, keepdims=True)
    code = lax.broadcasted_iota(jnp.int32, dist.shape, 0)
    idx = jnp.min(jnp.where(dist == minval, code, _K), axis=0,
                  keepdims=True)
    step = pl.program_id(0)
    idx_ref[pl.ds(step % 8, 1), :] = idx
    prev = loss_ref[...]
    acc = jnp.where(step == 0, jnp.zeros_like(prev), prev) + jnp.sum(minval)
    loss_ref[...] = jnp.where(step == pl.num_programs(0) - 1,
                              acc * _LOSS_SCALE, acc)


_vq_tc = pl.pallas_call(
    _vq_tc_body,
    grid=(_G,),
    in_specs=[
        pl.BlockSpec((1, _D, _T), lambda i: (i, 0, 0)),
        pl.BlockSpec((_K, _D), lambda i: (0, 0)),
    ],
    out_specs=[
        pl.BlockSpec((8, _T), lambda i: (i // 8, 0)),
        pl.BlockSpec((1, 1), lambda i: (0, 0)),
    ],
    out_shape=[
        jax.ShapeDtypeStruct((_N // _T, _T), jnp.int32),
        jax.ShapeDtypeStruct((1, 1), jnp.float32),
    ],
)


def _sc_gather_body(idx_hbm, w_hbm, out_hbm, idx_v, w_sp, rows_v,
                    gsem, wsem):
    wid = lax.axis_index("s") * _NC + lax.axis_index("c")
    sid = lax.axis_index("s")

    @pl.when(sid == 0)
    def _():
        pltpu.sync_copy(w_hbm, w_sp)
    plsc.subcore_barrier()
    pltpu.sync_copy(idx_hbm.at[pl.ds(wid * _TOK_PER_W, _TOK_PER_W)], idx_v)

    wh = [None, None]
    for g in range(_NCHUNK):
        b = g % 2
        if wh[b] is not None:
            wh[b].wait()
        pltpu.async_copy(w_sp.at[idx_v.at[pl.ds(g * _CH, _CH)]],
                         rows_v.at[b], gsem).wait()
        wh[b] = pltpu.async_copy(
            rows_v.at[b],
            out_hbm.at[pl.ds(wid * _TOK_PER_W + g * _CH, _CH)],
            wsem.at[b])
    for b in range(2):
        if wh[b] is not None:
            wh[b].wait()


@functools.cache
def _sc_gather():
    return pl.kernel(
        _sc_gather_body,
        mesh=plsc.VectorSubcoreMesh(core_axis_name="c", subcore_axis_name="s"),
        out_type=jax.ShapeDtypeStruct((_N, _D), jnp.float32),
        compiler_params=pltpu.CompilerParams(use_tc_tiling_on_sc=False),
        scratch_types=[
            pltpu.VMEM((_TOK_PER_W,), jnp.int32),
            pltpu.VMEM_SHARED((_K, _D), jnp.float32),
            pltpu.VMEM((2, _CH, _D), jnp.float32),
            pltpu.SemaphoreType.DMA,
            pltpu.SemaphoreType.DMA((2,)),
        ],
    )


def kernel(z, W):
    zt = z.transpose(0, 2, 1)
    w_m2 = W * (-2.0)
    idx_rows, loss_arr = _vq_tc(zt, w_m2)
    z_q = _sc_gather()(idx_rows.reshape(-1), W)
    return (z_q.reshape(z.shape), loss_arr[0, 0],
            idx_rows.reshape(z.shape[0], z.shape[1]))
